# baseline (speedup 1.0000x reference)
import jax
import jax.numpy as jnp
from jax import lax
from jax.experimental import pallas as pl


def _pos_kernel(col_ref, row_ref, out_ref):
    w, d = col_ref.shape
    h, _ = row_ref.shape
    hw = h * w
    col = col_ref[...]
    row = row_ref[...]
    i_idx = lax.broadcasted_iota(jnp.int32, (w, hw), 0)
    p_idx = lax.broadcasted_iota(jnp.int32, (w, hw), 1)
    sel_x = (p_idx % w == i_idx).astype(jnp.float32)
    j_idx = lax.broadcasted_iota(jnp.int32, (h, hw), 0)
    q_idx = lax.broadcasted_iota(jnp.int32, (h, hw), 1)
    sel_y = (q_idx // w == j_idx).astype(jnp.float32)
    x_part = lax.dot_general(col, sel_x, (((0,), (0,)), ((), ())),
                             preferred_element_type=jnp.float32)
    y_part = lax.dot_general(row, sel_y, (((0,), (0,)), ((), ())),
                             preferred_element_type=jnp.float32)
    out_ref[0, 0:d, :] = x_part
    out_ref[0, d:2 * d, :] = y_part


def kernel(tensor_list, row_embed, col_embed):
    b = tensor_list.shape[0]
    h, w = tensor_list.shape[-2], tensor_list.shape[-1]
    d = col_embed.shape[-1]
    out = pl.pallas_call(
        _pos_kernel,
        out_shape=jax.ShapeDtypeStruct((b, 2 * d, h * w), jnp.float32),
        grid=(b,),
        in_specs=[
            pl.BlockSpec((w, d), lambda i: (0, 0)),
            pl.BlockSpec((h, d), lambda i: (0, 0)),
        ],
        out_specs=pl.BlockSpec((1, 2 * d, h * w), lambda i: (i, 0, 0)),
    )(col_embed[:w], row_embed[:h])
    return out.reshape(b, 2 * d, h, w)


# speedup vs baseline: 1.1286x; 1.1286x over previous
import jax
import jax.numpy as jnp
from jax import lax
from jax.experimental import pallas as pl
from jax.experimental.pallas import tpu as pltpu


def _pos_kernel(col_ref, row_ref, out_ref, scratch, sem):
    w, d = col_ref.shape
    h, _ = row_ref.shape
    b = out_ref.shape[0]
    hw = h * w
    col = col_ref[...]
    row = row_ref[...]
    i_idx = lax.broadcasted_iota(jnp.int32, (w, hw), 0)
    p_idx = lax.broadcasted_iota(jnp.int32, (w, hw), 1)
    sel_x = (p_idx % w == i_idx).astype(jnp.float32)
    j_idx = lax.broadcasted_iota(jnp.int32, (h, hw), 0)
    q_idx = lax.broadcasted_iota(jnp.int32, (h, hw), 1)
    sel_y = (q_idx // w == j_idx).astype(jnp.float32)
    scratch[0:d, :] = lax.dot_general(col, sel_x, (((0,), (0,)), ((), ())),
                                      preferred_element_type=jnp.float32)
    scratch[d:2 * d, :] = lax.dot_general(row, sel_y, (((0,), (0,)), ((), ())),
                                          preferred_element_type=jnp.float32)
    for i in range(b):
        pltpu.make_async_copy(scratch, out_ref.at[i], sem.at[i]).start()
    for i in range(b):
        pltpu.make_async_copy(scratch, out_ref.at[i], sem.at[i]).wait()


def kernel(tensor_list, row_embed, col_embed):
    b = tensor_list.shape[0]
    h, w = tensor_list.shape[-2], tensor_list.shape[-1]
    d = col_embed.shape[-1]
    out = pl.pallas_call(
        _pos_kernel,
        out_shape=jax.ShapeDtypeStruct((b, 2 * d, h * w), jnp.float32),
        out_specs=pl.BlockSpec(memory_space=pl.ANY),
        scratch_shapes=[
            pltpu.VMEM((2 * d, h * w), jnp.float32),
            pltpu.SemaphoreType.DMA((b,)),
        ],
    )(col_embed[:w], row_embed[:h])
    return out.reshape(b, 2 * d, h, w)
